# baseline (speedup 1.0000x reference)
import functools

import jax
import jax.numpy as jnp
from jax import lax
from jax.experimental import pallas as pl
from jax.experimental.pallas import tpu as pltpu
from jax.experimental.pallas import tpu_sc as plsc

N = 10000
NP = 10240
E = 320000
D_IN = 128
D_HID = 128
N_CLS = 16

B = 128
NC = 2
NS = 16
NW = NC * NS
ROWS_PER_TILE = 80
NROWS = NW * ROWS_PER_TILE
EP = NROWS * B
DEG_ROWS_PER_TILE = NROWS // NS
NPT = NP // NS
R = 1000

_f32 = jnp.float32
_mesh = plsc.VectorSubcoreMesh(core_axis_name="c", subcore_axis_name="s")


@functools.partial(
    pl.kernel,
    out_type=jax.ShapeDtypeStruct((2, NP), _f32),
    mesh=_mesh,
    scratch_types=[
        pltpu.VMEM((DEG_ROWS_PER_TILE, B), jnp.int32),
        pltpu.VMEM((DEG_ROWS_PER_TILE, B), _f32),
        pltpu.VMEM_SHARED((NP,), _f32),
    ],
)
def _sc_degrees(e3d_hbm, wgt_hbm, zeros_hbm, out_hbm, idx_v, wgt_v, acc_sp):
    c = lax.axis_index("c")
    s = lax.axis_index("s")

    rsl = pl.ds(s * DEG_ROWS_PER_TILE, DEG_ROWS_PER_TILE)
    pltpu.sync_copy(e3d_hbm.at[c].at[rsl], idx_v)
    pltpu.sync_copy(wgt_hbm.at[rsl], wgt_v)

    @pl.when(s == 0)
    def _():
        pltpu.sync_copy(zeros_hbm, acc_sp)

    plsc.subcore_barrier()

    @pl.loop(0, DEG_ROWS_PER_TILE)
    def _(j):
        pltpu.sync_copy(wgt_v.at[j], acc_sp.at[idx_v.at[j]], add=True)

    plsc.subcore_barrier()

    sl = pl.ds(s * NPT, NPT)
    pltpu.sync_copy(acc_sp.at[sl], out_hbm.at[c].at[sl])


def _make_mp(D):
    half = ROWS_PER_TILE // 2

    @functools.partial(
        pl.kernel,
        out_type=jax.ShapeDtypeStruct((NC, NP, D), _f32),
        mesh=_mesh,
        scratch_types=[
            pltpu.VMEM((half, B), jnp.int32),
            pltpu.VMEM((half, B), jnp.int32),
            pltpu.VMEM((B, D), _f32),
            pltpu.VMEM((B, D), _f32),
            pltpu.VMEM_SHARED((NP, D), _f32),
            pltpu.SemaphoreType.DMA,
            pltpu.SemaphoreType.DMA,
            pltpu.SemaphoreType.DMA,
            pltpu.SemaphoreType.DMA,
        ],
    )
    def _mp(src2d, dst2d, h_hbm, zeros_hbm, out_hbm,
            src_v, dst_v, rows_a, rows_b, acc_sp, sga, sgb, ssa, ssb):
        c = lax.axis_index("c")
        s = lax.axis_index("s")
        wid = s * NC + c

        nsl = pl.ds(s * NPT, NPT)
        pltpu.sync_copy(zeros_hbm.at[nsl], acc_sp.at[nsl])
        plsc.subcore_barrier()

        def g_start(buf, sem, j):
            pltpu.make_async_copy(h_hbm.at[src_v.at[j]], buf, sem).start()

        def g_wait(buf, sem, j):
            pltpu.make_async_copy(h_hbm.at[src_v.at[j]], buf, sem).wait()

        def s_start(buf, sem, j):
            pltpu.make_async_copy(buf, acc_sp.at[dst_v.at[j]], sem).start(add=True)

        def s_wait(buf, sem, j):
            pltpu.make_async_copy(buf, acc_sp.at[dst_v.at[j]], sem).wait()

        for h in range(2):
            rsl = pl.ds(wid * ROWS_PER_TILE + h * half, half)
            pltpu.sync_copy(src2d.at[rsl], src_v)
            pltpu.sync_copy(dst2d.at[rsl], dst_v)

            g_start(rows_a, sga, 0)
            g_start(rows_b, sgb, 1)

            @pl.loop(0, half, step=2)
            def _(j):
                g_wait(rows_a, sga, j)
                s_start(rows_a, ssa, j)
                g_wait(rows_b, sgb, j + 1)
                s_start(rows_b, ssb, j + 1)
                s_wait(rows_a, ssa, j)

                @pl.when(j + 2 < half)
                def _():
                    g_start(rows_a, sga, j + 2)

                s_wait(rows_b, ssb, j + 1)

                @pl.when(j + 3 < half)
                def _():
                    g_start(rows_b, sgb, j + 3)

        plsc.subcore_barrier()
        pltpu.sync_copy(acc_sp.at[nsl], out_hbm.at[c].at[nsl])

    return _mp


_mp128 = _make_mp(D_HID)


def _norm(d):
    return jnp.where(d > 0, lax.rsqrt(jnp.maximum(d, 1.0)), 0.0)


def _tc_prep_body(deg_o_ref, x_ref, w_ref, out_ref):
    no = _norm(deg_o_ref[...])
    y = jnp.dot(x_ref[...], w_ref[...], preferred_element_type=_f32)
    out_ref[...] = y * no


_tc_prep = pl.pallas_call(
    _tc_prep_body,
    grid=(N // R,),
    in_specs=[
        pl.BlockSpec((R, 1), lambda i: (i, 0)),
        pl.BlockSpec((R, D_IN), lambda i: (i, 0)),
        pl.BlockSpec((D_IN, D_HID), lambda i: (0, 0)),
    ],
    out_specs=pl.BlockSpec((R, D_HID), lambda i: (i, 0)),
    out_shape=jax.ShapeDtypeStruct((N, D_HID), _f32),
)


def _tc_mid_body(parts_ref, deg_i_ref, deg_o_ref, b1_ref, w2_ref, out_ref):
    ni = _norm(deg_i_ref[...])
    no = _norm(deg_o_ref[...])
    ps = parts_ref[...]
    agg = ps[0] + ps[1]
    h = jnp.maximum(agg * ni + b1_ref[...], 0.0)
    y = jnp.dot(h, w2_ref[...], preferred_element_type=_f32)
    out_ref[...] = jnp.concatenate(
        [y * no, jnp.zeros((y.shape[0], D_HID - N_CLS), _f32)], axis=1)


_tc_mid = pl.pallas_call(
    _tc_mid_body,
    grid=(N // R,),
    in_specs=[
        pl.BlockSpec((NC, R, D_HID), lambda i: (0, i, 0)),
        pl.BlockSpec((R, 1), lambda i: (i, 0)),
        pl.BlockSpec((R, 1), lambda i: (i, 0)),
        pl.BlockSpec((1, D_HID), lambda i: (0, 0)),
        pl.BlockSpec((D_HID, N_CLS), lambda i: (0, 0)),
    ],
    out_specs=pl.BlockSpec((R, D_HID), lambda i: (i, 0)),
    out_shape=jax.ShapeDtypeStruct((NP, D_HID), _f32),
)


def _tc_out_body(parts_ref, deg_i_ref, b2_ref, out_ref):
    ni = _norm(deg_i_ref[...])
    ps = parts_ref[...]
    out_ref[...] = (ps[0, :, :N_CLS] + ps[1, :, :N_CLS]) * ni + b2_ref[...]


_tc_out = pl.pallas_call(
    _tc_out_body,
    grid=(N // R,),
    in_specs=[
        pl.BlockSpec((NC, R, D_HID), lambda i: (0, i, 0)),
        pl.BlockSpec((R, 1), lambda i: (i, 0)),
        pl.BlockSpec((1, N_CLS), lambda i: (0, 0)),
    ],
    out_specs=pl.BlockSpec((R, N_CLS), lambda i: (i, 0)),
    out_shape=jax.ShapeDtypeStruct((N, N_CLS), _f32),
)


def kernel(x, edge_index, W1, b1, W2, b2):
    pad = EP - E
    iota = jnp.arange(pad, dtype=jnp.int32)
    src_pad = (iota * 37) % N
    dst_pad = N + iota % (NP - N)
    srcp = jnp.concatenate([edge_index[0], src_pad]).reshape(NROWS, B)
    dstp = jnp.concatenate([edge_index[1], dst_pad]).reshape(NROWS, B)
    e3d = jnp.stack([srcp, dstp])
    wgt = jnp.concatenate(
        [jnp.ones((E,), _f32), jnp.zeros((pad,), _f32)]
    ).reshape(NROWS, B)
    zeros_n = jnp.zeros((NP,), _f32)
    zeros_nd = jnp.zeros((NP, D_HID), _f32)

    deg = _sc_degrees(e3d, wgt, zeros_n)
    deg_o = deg[0, :N].reshape(N, 1)
    deg_i = deg[1, :N].reshape(N, 1)

    h1s = _tc_prep(deg_o, x, W1)
    parts = _mp128(srcp, dstp, h1s, zeros_nd)
    h2s = _tc_mid(parts, deg_i, deg_o, b1.reshape(1, D_HID), W2)
    parts2 = _mp128(srcp, dstp, h2s, zeros_nd)
    return _tc_out(parts2, deg_i, b2.reshape(1, N_CLS))


# speedup vs baseline: 1.0858x; 1.0858x over previous
import functools

import jax
import jax.numpy as jnp
from jax import lax
from jax.experimental import pallas as pl
from jax.experimental.pallas import tpu as pltpu
from jax.experimental.pallas import tpu_sc as plsc

N = 10000
NP = 10240
E = 320000
D_IN = 128
D_HID = 128
N_CLS = 16

B = 64
NC = 2
NS = 16
NW = NC * NS
NROWS = E // B
MP_RPT = 160
MP_TAIL = NROWS - (NW - 1) * MP_RPT
DEG_RPT = 312
DEG_TAIL = NROWS - (NS - 1) * DEG_RPT
NPT = NP // NS
R = 1000

_f32 = jnp.float32
_mesh = plsc.VectorSubcoreMesh(core_axis_name="c", subcore_axis_name="s")


@functools.partial(
    pl.kernel,
    out_type=jax.ShapeDtypeStruct((2, NP), _f32),
    mesh=_mesh,
    scratch_types=[
        pltpu.VMEM((DEG_TAIL, B), jnp.int32),
        pltpu.VMEM((B,), _f32),
        pltpu.VMEM_SHARED((NP,), _f32),
    ],
)
def _sc_degrees(e3d_hbm, zeros_hbm, out_hbm, idx_v, ones_v, acc_sp):
    c = lax.axis_index("c")
    s = lax.axis_index("s")

    @pl.loop(0, B, step=16)
    def _(i):
        ones_v[pl.ds(i, 16)] = jnp.full((16,), 1.0, _f32)

    @pl.when(s == 0)
    def _():
        pltpu.sync_copy(zeros_hbm, acc_sp)

    @pl.when(s < NS - 1)
    def _():
        pltpu.sync_copy(e3d_hbm.at[c].at[pl.ds(s * DEG_RPT, DEG_RPT)],
                        idx_v.at[pl.ds(0, DEG_RPT)])

    @pl.when(s == NS - 1)
    def _():
        pltpu.sync_copy(e3d_hbm.at[c].at[pl.ds(s * DEG_RPT, DEG_TAIL)], idx_v)

    plsc.subcore_barrier()

    @pl.when(s < NS - 1)
    def _():
        @pl.loop(0, DEG_RPT)
        def _(j):
            pltpu.sync_copy(ones_v, acc_sp.at[idx_v.at[j]], add=True)

    @pl.when(s == NS - 1)
    def _():
        @pl.loop(0, DEG_TAIL)
        def _(j):
            pltpu.sync_copy(ones_v, acc_sp.at[idx_v.at[j]], add=True)

    plsc.subcore_barrier()

    sl = pl.ds(s * NPT, NPT)
    pltpu.sync_copy(acc_sp.at[sl], out_hbm.at[c].at[sl])


def _make_mp(D):
    @functools.partial(
        pl.kernel,
        out_type=jax.ShapeDtypeStruct((NC, NP, D), _f32),
        mesh=_mesh,
        scratch_types=[
            pltpu.VMEM((MP_RPT // 2, B), jnp.int32),
            pltpu.VMEM((MP_RPT // 2, B), jnp.int32),
            pltpu.VMEM((B, D), _f32),
            pltpu.VMEM((B, D), _f32),
            pltpu.VMEM_SHARED((NP, D), _f32),
            pltpu.SemaphoreType.DMA,
            pltpu.SemaphoreType.DMA,
        ],
    )
    def _mp(e3d_hbm, h_hbm, zeros_hbm, out_hbm,
            src_v, dst_v, rows_a, rows_b, acc_sp, sem_a, sem_b):
        c = lax.axis_index("c")
        s = lax.axis_index("s")
        wid = s * NC + c

        nsl = pl.ds(s * NPT, NPT)
        pltpu.sync_copy(zeros_hbm.at[nsl], acc_sp.at[nsl])
        plsc.subcore_barrier()

        def run_chunk(row0, nb):
            rsl = pl.ds(row0, nb)
            bsl = pl.ds(0, nb)
            pltpu.sync_copy(e3d_hbm.at[0].at[rsl], src_v.at[bsl])
            pltpu.sync_copy(e3d_hbm.at[1].at[rsl], dst_v.at[bsl])
            pltpu.make_async_copy(h_hbm.at[src_v.at[0]], rows_a, sem_a).start()

            @pl.loop(0, nb, step=2)
            def _(j):
                pltpu.make_async_copy(h_hbm.at[src_v.at[j + 1]], rows_b, sem_b).start()
                pltpu.make_async_copy(h_hbm.at[src_v.at[j]], rows_a, sem_a).wait()
                pltpu.sync_copy(rows_a, acc_sp.at[dst_v.at[j]], add=True)

                @pl.when(j + 2 < nb)
                def _():
                    pltpu.make_async_copy(h_hbm.at[src_v.at[j + 2]], rows_a, sem_a).start()

                pltpu.make_async_copy(h_hbm.at[src_v.at[j + 1]], rows_b, sem_b).wait()
                pltpu.sync_copy(rows_b, acc_sp.at[dst_v.at[j + 1]], add=True)

        @pl.when(wid < NW - 1)
        def _():
            run_chunk(wid * MP_RPT, MP_RPT // 2)
            run_chunk(wid * MP_RPT + MP_RPT // 2, MP_RPT // 2)

        @pl.when(wid == NW - 1)
        def _():
            run_chunk(wid * MP_RPT, MP_TAIL)

        plsc.subcore_barrier()
        pltpu.sync_copy(acc_sp.at[nsl], out_hbm.at[c].at[nsl])

    return _mp


_mp128 = _make_mp(D_HID)


def _norm(d):
    return jnp.where(d > 0, lax.rsqrt(jnp.maximum(d, 1.0)), 0.0)


def _tc_prep_body(deg_o_ref, x_ref, w_ref, out_ref):
    no = _norm(deg_o_ref[...])
    y = jnp.dot(x_ref[...], w_ref[...], preferred_element_type=_f32)
    out_ref[...] = y * no


_tc_prep = pl.pallas_call(
    _tc_prep_body,
    grid=(N // R,),
    in_specs=[
        pl.BlockSpec((R, 1), lambda i: (i, 0)),
        pl.BlockSpec((R, D_IN), lambda i: (i, 0)),
        pl.BlockSpec((D_IN, D_HID), lambda i: (0, 0)),
    ],
    out_specs=pl.BlockSpec((R, D_HID), lambda i: (i, 0)),
    out_shape=jax.ShapeDtypeStruct((N, D_HID), _f32),
)


def _tc_mid_body(parts_ref, deg_i_ref, deg_o_ref, b1_ref, w2_ref, out_ref):
    ni = _norm(deg_i_ref[...])
    no = _norm(deg_o_ref[...])
    ps = parts_ref[...]
    agg = ps[0] + ps[1]
    h = jnp.maximum(agg * ni + b1_ref[...], 0.0)
    y = jnp.dot(h, w2_ref[...], preferred_element_type=_f32)
    out_ref[...] = jnp.concatenate(
        [y * no, jnp.zeros((y.shape[0], D_HID - N_CLS), _f32)], axis=1)


_tc_mid = pl.pallas_call(
    _tc_mid_body,
    grid=(N // R,),
    in_specs=[
        pl.BlockSpec((NC, R, D_HID), lambda i: (0, i, 0)),
        pl.BlockSpec((R, 1), lambda i: (i, 0)),
        pl.BlockSpec((R, 1), lambda i: (i, 0)),
        pl.BlockSpec((1, D_HID), lambda i: (0, 0)),
        pl.BlockSpec((D_HID, N_CLS), lambda i: (0, 0)),
    ],
    out_specs=pl.BlockSpec((R, D_HID), lambda i: (i, 0)),
    out_shape=jax.ShapeDtypeStruct((NP, D_HID), _f32),
)


def _tc_out_body(parts_ref, deg_i_ref, b2_ref, out_ref):
    ni = _norm(deg_i_ref[...])
    ps = parts_ref[...]
    out_ref[...] = (ps[0, :, :N_CLS] + ps[1, :, :N_CLS]) * ni + b2_ref[...]


_tc_out = pl.pallas_call(
    _tc_out_body,
    grid=(N // R,),
    in_specs=[
        pl.BlockSpec((NC, R, D_HID), lambda i: (0, i, 0)),
        pl.BlockSpec((R, 1), lambda i: (i, 0)),
        pl.BlockSpec((1, N_CLS), lambda i: (0, 0)),
    ],
    out_specs=pl.BlockSpec((R, N_CLS), lambda i: (i, 0)),
    out_shape=jax.ShapeDtypeStruct((N, N_CLS), _f32),
)


def kernel(x, edge_index, W1, b1, W2, b2):
    e3d = edge_index.reshape(2, NROWS, B)
    zeros_n = jnp.zeros((NP,), _f32)
    zeros_nd = jnp.zeros((NP, D_HID), _f32)

    deg = _sc_degrees(e3d, zeros_n)
    deg_o = deg[0, :N].reshape(N, 1)
    deg_i = deg[1, :N].reshape(N, 1)

    h1s = _tc_prep(deg_o, x, W1)
    parts = _mp128(e3d, h1s, zeros_nd)
    h2s = _tc_mid(parts, deg_i, deg_o, b1.reshape(1, D_HID), W2)
    parts2 = _mp128(e3d, h2s, zeros_nd)
    return _tc_out(parts2, deg_i, b2.reshape(1, N_CLS))


# speedup vs baseline: 1.2225x; 1.1259x over previous
import functools

import jax
import jax.numpy as jnp
from jax import lax
from jax.experimental import pallas as pl
from jax.experimental.pallas import tpu as pltpu
from jax.experimental.pallas import tpu_sc as plsc

N = 10000
NP = 10240
E = 320000
D_IN = 128
D_HID = 128
N_CLS = 16

B = 128
NC = 2
NS = 16
NW = NC * NS
NR128 = E // 128
NR64 = E // 64
MP_RPT = 80
MP_TAIL64 = 40
DEG_RPT = 160
DEG_TAIL64 = 200
NPT = NP // NS
R = 1000

_f32 = jnp.float32
_mesh = plsc.VectorSubcoreMesh(core_axis_name="c", subcore_axis_name="s")


@functools.partial(
    pl.kernel,
    out_type=jax.ShapeDtypeStruct((2, NP), _f32),
    mesh=_mesh,
    scratch_types=[
        pltpu.VMEM((DEG_RPT, 128), jnp.int32),
        pltpu.VMEM((DEG_TAIL64, 64), jnp.int32),
        pltpu.VMEM((128,), _f32),
        pltpu.VMEM_SHARED((NP,), _f32),
    ],
)
def _sc_degrees(e128_hbm, e64_hbm, zeros_hbm, out_hbm, idx_v, idx64_v, ones_v, acc_sp):
    c = lax.axis_index("c")
    s = lax.axis_index("s")

    @pl.loop(0, 128, step=16)
    def _(i):
        ones_v[pl.ds(i, 16)] = jnp.full((16,), 1.0, _f32)

    @pl.when(s == 0)
    def _():
        pltpu.sync_copy(zeros_hbm, acc_sp)

    @pl.when(s < NS - 1)
    def _():
        pltpu.sync_copy(e128_hbm.at[c].at[pl.ds(s * DEG_RPT, DEG_RPT)], idx_v)

    @pl.when(s == NS - 1)
    def _():
        pltpu.sync_copy(e64_hbm.at[c].at[pl.ds(NR64 - DEG_TAIL64, DEG_TAIL64)],
                        idx64_v)

    plsc.subcore_barrier()

    @pl.when(s < NS - 1)
    def _():
        @pl.loop(0, DEG_RPT)
        def _(j):
            pltpu.sync_copy(ones_v, acc_sp.at[idx_v.at[j]], add=True)

    @pl.when(s == NS - 1)
    def _():
        @pl.loop(0, DEG_TAIL64)
        def _(j):
            pltpu.sync_copy(ones_v.at[pl.ds(0, 64)], acc_sp.at[idx64_v.at[j]],
                            add=True)

    plsc.subcore_barrier()

    sl = pl.ds(s * NPT, NPT)
    pltpu.sync_copy(acc_sp.at[sl], out_hbm.at[c].at[sl])


def _make_mp(D):
    @functools.partial(
        pl.kernel,
        out_type=jax.ShapeDtypeStruct((NC, NP, D), _f32),
        mesh=_mesh,
        scratch_types=[
            pltpu.VMEM((24, B), jnp.int32),
            pltpu.VMEM((24, B), jnp.int32),
            pltpu.VMEM((MP_TAIL64, 64), jnp.int32),
            pltpu.VMEM((MP_TAIL64, 64), jnp.int32),
            pltpu.VMEM((B, D), _f32),
            pltpu.VMEM((B, D), _f32),
            pltpu.VMEM_SHARED((NP, D), _f32),
            pltpu.SemaphoreType.DMA,
            pltpu.SemaphoreType.DMA,
        ],
    )
    def _mp(e128_hbm, e64_hbm, h_hbm, zeros_hbm, out_hbm,
            src_v, dst_v, src64_v, dst64_v, rows_a, rows_b, acc_sp, sem_a, sem_b):
        c = lax.axis_index("c")
        s = lax.axis_index("s")
        wid = s * NC + c

        nsl = pl.ds(s * NPT, NPT)
        pltpu.sync_copy(zeros_hbm.at[nsl], acc_sp.at[nsl])
        plsc.subcore_barrier()

        def pipeline(nb, src_idx, dst_idx, buf_a, buf_b):
            pltpu.make_async_copy(h_hbm.at[src_idx.at[0]], buf_a, sem_a).start()

            @pl.loop(0, nb, step=2)
            def _(j):
                pltpu.make_async_copy(h_hbm.at[src_idx.at[j + 1]], buf_b, sem_b).start()
                pltpu.make_async_copy(h_hbm.at[src_idx.at[j]], buf_a, sem_a).wait()
                pltpu.sync_copy(buf_a, acc_sp.at[dst_idx.at[j]], add=True)

                @pl.when(j + 2 < nb)
                def _():
                    pltpu.make_async_copy(h_hbm.at[src_idx.at[j + 2]], buf_a, sem_a).start()

                pltpu.make_async_copy(h_hbm.at[src_idx.at[j + 1]], buf_b, sem_b).wait()
                pltpu.sync_copy(buf_b, acc_sp.at[dst_idx.at[j + 1]], add=True)

        @pl.when(wid < NW - 1)
        def _():
            row0 = wid * MP_RPT
            for nb in (24, 24, 24, 8):
                rsl = pl.ds(row0, nb)
                bsl = pl.ds(0, nb)
                pltpu.sync_copy(e128_hbm.at[0].at[rsl], src_v.at[bsl])
                pltpu.sync_copy(e128_hbm.at[1].at[rsl], dst_v.at[bsl])
                pipeline(nb, src_v, dst_v, rows_a, rows_b)
                row0 += nb

        @pl.when(wid == NW - 1)
        def _():
            rsl = pl.ds(NR64 - MP_TAIL64, MP_TAIL64)
            pltpu.sync_copy(e64_hbm.at[0].at[rsl], src64_v)
            pltpu.sync_copy(e64_hbm.at[1].at[rsl], dst64_v)
            buf = rows_a.at[pl.ds(0, 64)]

            @pl.loop(0, MP_TAIL64)
            def _(j):
                pltpu.sync_copy(h_hbm.at[src64_v.at[j]], buf)
                pltpu.sync_copy(buf, acc_sp.at[dst64_v.at[j]], add=True)

        plsc.subcore_barrier()
        pltpu.sync_copy(acc_sp.at[nsl], out_hbm.at[c].at[nsl])

    return _mp


_mp128 = _make_mp(D_HID)


def _norm(d):
    return jnp.where(d > 0, lax.rsqrt(jnp.maximum(d, 1.0)), 0.0)


def _tc_prep_body(deg_o_ref, x_ref, w_ref, out_ref):
    no = _norm(deg_o_ref[...])
    y = jnp.dot(x_ref[...], w_ref[...], preferred_element_type=_f32)
    out_ref[...] = y * no


_tc_prep = pl.pallas_call(
    _tc_prep_body,
    grid=(N // R,),
    in_specs=[
        pl.BlockSpec((R, 1), lambda i: (i, 0)),
        pl.BlockSpec((R, D_IN), lambda i: (i, 0)),
        pl.BlockSpec((D_IN, D_HID), lambda i: (0, 0)),
    ],
    out_specs=pl.BlockSpec((R, D_HID), lambda i: (i, 0)),
    out_shape=jax.ShapeDtypeStruct((N, D_HID), _f32),
)


def _tc_mid_body(parts_ref, deg_i_ref, deg_o_ref, b1_ref, w2_ref, out_ref):
    ni = _norm(deg_i_ref[...])
    no = _norm(deg_o_ref[...])
    ps = parts_ref[...]
    agg = ps[0] + ps[1]
    h = jnp.maximum(agg * ni + b1_ref[...], 0.0)
    y = jnp.dot(h, w2_ref[...], preferred_element_type=_f32)
    out_ref[...] = jnp.concatenate(
        [y * no, jnp.zeros((y.shape[0], D_HID - N_CLS), _f32)], axis=1)


_tc_mid = pl.pallas_call(
    _tc_mid_body,
    grid=(N // R,),
    in_specs=[
        pl.BlockSpec((NC, R, D_HID), lambda i: (0, i, 0)),
        pl.BlockSpec((R, 1), lambda i: (i, 0)),
        pl.BlockSpec((R, 1), lambda i: (i, 0)),
        pl.BlockSpec((1, D_HID), lambda i: (0, 0)),
        pl.BlockSpec((D_HID, N_CLS), lambda i: (0, 0)),
    ],
    out_specs=pl.BlockSpec((R, D_HID), lambda i: (i, 0)),
    out_shape=jax.ShapeDtypeStruct((NP, D_HID), _f32),
)


def _tc_out_body(parts_ref, deg_i_ref, b2_ref, out_ref):
    ni = _norm(deg_i_ref[...])
    ps = parts_ref[...]
    out_ref[...] = (ps[0, :, :N_CLS] + ps[1, :, :N_CLS]) * ni + b2_ref[...]


_tc_out = pl.pallas_call(
    _tc_out_body,
    grid=(N // R,),
    in_specs=[
        pl.BlockSpec((NC, R, D_HID), lambda i: (0, i, 0)),
        pl.BlockSpec((R, 1), lambda i: (i, 0)),
        pl.BlockSpec((1, N_CLS), lambda i: (0, 0)),
    ],
    out_specs=pl.BlockSpec((R, N_CLS), lambda i: (i, 0)),
    out_shape=jax.ShapeDtypeStruct((N, N_CLS), _f32),
)


def kernel(x, edge_index, W1, b1, W2, b2):
    e128 = edge_index.reshape(2, NR128, 128)
    e64 = edge_index.reshape(2, NR64, 64)
    zeros_n = jnp.zeros((NP,), _f32)
    zeros_nd = jnp.zeros((NP, D_HID), _f32)

    deg = _sc_degrees(e128, e64, zeros_n)
    deg_o = deg[0, :N].reshape(N, 1)
    deg_i = deg[1, :N].reshape(N, 1)

    h1s = _tc_prep(deg_o, x, W1)
    parts = _mp128(e128, e64, h1s, zeros_nd)
    h2s = _tc_mid(parts, deg_i, deg_o, b1.reshape(1, D_HID), W2)
    parts2 = _mp128(e128, e64, h2s, zeros_nd)
    return _tc_out(parts2, deg_i, b2.reshape(1, N_CLS))


# speedup vs baseline: 1.2276x; 1.0042x over previous
import functools

import jax
import jax.numpy as jnp
from jax import lax
from jax.experimental import pallas as pl
from jax.experimental.pallas import tpu as pltpu
from jax.experimental.pallas import tpu_sc as plsc

N = 10000
NP = 10240
E = 320000
D_IN = 128
D_HID = 128
N_CLS = 16

B = 128
NC = 2
NS = 16
NW = NC * NS
NR128 = E // 128
NR64 = E // 64
MP_RPT = 80
MP_TAIL64 = 40
DEG_RPT = 160
DEG_TAIL64 = 200
NPT = NP // NS
R = 1000

_f32 = jnp.float32
_mesh = plsc.VectorSubcoreMesh(core_axis_name="c", subcore_axis_name="s")


@functools.partial(
    pl.kernel,
    out_type=jax.ShapeDtypeStruct((2, NP), _f32),
    mesh=_mesh,
    scratch_types=[
        pltpu.VMEM((DEG_RPT, 128), jnp.int32),
        pltpu.VMEM((DEG_TAIL64, 64), jnp.int32),
        pltpu.VMEM((128,), _f32),
        pltpu.VMEM_SHARED((NP,), _f32),
    ],
)
def _sc_degrees(e128_hbm, e64_hbm, zeros_hbm, out_hbm, idx_v, idx64_v, ones_v, acc_sp):
    c = lax.axis_index("c")
    s = lax.axis_index("s")

    @pl.loop(0, 128, step=16)
    def _(i):
        ones_v[pl.ds(i, 16)] = jnp.full((16,), 1.0, _f32)

    @pl.when(s == 0)
    def _():
        pltpu.sync_copy(zeros_hbm, acc_sp)

    @pl.when(s < NS - 1)
    def _():
        pltpu.sync_copy(e128_hbm.at[c].at[pl.ds(s * DEG_RPT, DEG_RPT)], idx_v)

    @pl.when(s == NS - 1)
    def _():
        pltpu.sync_copy(e64_hbm.at[c].at[pl.ds(NR64 - DEG_TAIL64, DEG_TAIL64)],
                        idx64_v)

    plsc.subcore_barrier()

    @pl.when(s < NS - 1)
    def _():
        @pl.loop(0, DEG_RPT)
        def _(j):
            pltpu.sync_copy(ones_v, acc_sp.at[idx_v.at[j]], add=True)

    @pl.when(s == NS - 1)
    def _():
        @pl.loop(0, DEG_TAIL64)
        def _(j):
            pltpu.sync_copy(ones_v.at[pl.ds(0, 64)], acc_sp.at[idx64_v.at[j]],
                            add=True)

    plsc.subcore_barrier()

    sl = pl.ds(s * NPT, NPT)
    pltpu.sync_copy(acc_sp.at[sl], out_hbm.at[c].at[sl])


def _make_mp(D):
    @functools.partial(
        pl.kernel,
        out_type=jax.ShapeDtypeStruct((NC, NP, D), _f32),
        mesh=_mesh,
        scratch_types=[
            pltpu.VMEM((24, B), jnp.int32),
            pltpu.VMEM((24, B), jnp.int32),
            pltpu.VMEM((MP_TAIL64, 64), jnp.int32),
            pltpu.VMEM((MP_TAIL64, 64), jnp.int32),
            pltpu.VMEM((B, D), _f32),
            pltpu.VMEM((B, D), _f32),
            pltpu.VMEM_SHARED((NP, D), _f32),
            pltpu.SemaphoreType.DMA,
            pltpu.SemaphoreType.DMA,
        ],
    )
    def _mp(e128_hbm, e64_hbm, h_hbm, zeros_hbm, out_hbm,
            src_v, dst_v, src64_v, dst64_v, rows_a, rows_b, acc_sp, sem_a, sem_b):
        c = lax.axis_index("c")
        s = lax.axis_index("s")
        wid = s * NC + c

        nsl = pl.ds(s * NPT, NPT)
        pltpu.sync_copy(zeros_hbm.at[nsl], acc_sp.at[nsl])
        plsc.subcore_barrier()

        def pipeline(nb, src_idx, dst_idx, buf_a, buf_b):
            pltpu.make_async_copy(h_hbm.at[src_idx.at[0]], buf_a, sem_a).start()

            @pl.loop(0, nb, step=2)
            def _(j):
                pltpu.make_async_copy(h_hbm.at[src_idx.at[j + 1]], buf_b, sem_b).start()
                pltpu.make_async_copy(h_hbm.at[src_idx.at[j]], buf_a, sem_a).wait()
                pltpu.sync_copy(buf_a, acc_sp.at[dst_idx.at[j]], add=True)

                @pl.when(j + 2 < nb)
                def _():
                    pltpu.make_async_copy(h_hbm.at[src_idx.at[j + 2]], buf_a, sem_a).start()

                pltpu.make_async_copy(h_hbm.at[src_idx.at[j + 1]], buf_b, sem_b).wait()
                pltpu.sync_copy(buf_b, acc_sp.at[dst_idx.at[j + 1]], add=True)

        @pl.when(wid < NW - 1)
        def _():
            row0 = wid * MP_RPT
            for nb in (24, 24, 24, 8):
                rsl = pl.ds(row0, nb)
                bsl = pl.ds(0, nb)
                pltpu.sync_copy(e128_hbm.at[0].at[rsl], src_v.at[bsl])
                pltpu.sync_copy(e128_hbm.at[1].at[rsl], dst_v.at[bsl])
                pipeline(nb, src_v, dst_v, rows_a, rows_b)
                row0 += nb

        @pl.when(wid == NW - 1)
        def _():
            rsl = pl.ds(NR64 - MP_TAIL64, MP_TAIL64)
            pltpu.sync_copy(e64_hbm.at[0].at[rsl], src64_v)
            pltpu.sync_copy(e64_hbm.at[1].at[rsl], dst64_v)
            buf = rows_a.at[pl.ds(0, 64)]

            @pl.loop(0, MP_TAIL64)
            def _(j):
                pltpu.sync_copy(h_hbm.at[src64_v.at[j]], buf)
                pltpu.sync_copy(buf, acc_sp.at[dst64_v.at[j]], add=True)

        plsc.subcore_barrier()
        pltpu.sync_copy(acc_sp.at[nsl], out_hbm.at[c].at[nsl])

    return _mp


_mp128 = _make_mp(D_HID)


def _norm(d):
    return jnp.where(d > 0, lax.rsqrt(jnp.maximum(d, 1.0)), 0.0)


def _tc_mm_body(x_ref, w_ref, out_ref):
    out_ref[...] = jnp.dot(x_ref[...], w_ref[...], preferred_element_type=_f32)


_tc_mm = pl.pallas_call(
    _tc_mm_body,
    grid=(N // R,),
    in_specs=[
        pl.BlockSpec((R, D_IN), lambda i: (i, 0)),
        pl.BlockSpec((D_IN, D_HID), lambda i: (0, 0)),
    ],
    out_specs=pl.BlockSpec((R, D_HID), lambda i: (i, 0)),
    out_shape=jax.ShapeDtypeStruct((N, D_HID), _f32),
)


def _tc_scale_body(deg_o_ref, y_ref, out_ref):
    out_ref[...] = y_ref[...] * _norm(deg_o_ref[...])


_tc_scale = pl.pallas_call(
    _tc_scale_body,
    grid=(N // R,),
    in_specs=[
        pl.BlockSpec((R, 1), lambda i: (i, 0)),
        pl.BlockSpec((R, D_HID), lambda i: (i, 0)),
    ],
    out_specs=pl.BlockSpec((R, D_HID), lambda i: (i, 0)),
    out_shape=jax.ShapeDtypeStruct((N, D_HID), _f32),
)


def _tc_mid_body(parts_ref, deg_i_ref, deg_o_ref, b1_ref, w2_ref, out_ref):
    ni = _norm(deg_i_ref[...])
    no = _norm(deg_o_ref[...])
    ps = parts_ref[...]
    agg = ps[0] + ps[1]
    h = jnp.maximum(agg * ni + b1_ref[...], 0.0)
    y = jnp.dot(h, w2_ref[...], preferred_element_type=_f32)
    out_ref[...] = jnp.concatenate(
        [y * no, jnp.zeros((y.shape[0], D_HID - N_CLS), _f32)], axis=1)


_tc_mid = pl.pallas_call(
    _tc_mid_body,
    grid=(N // R,),
    in_specs=[
        pl.BlockSpec((NC, R, D_HID), lambda i: (0, i, 0)),
        pl.BlockSpec((R, 1), lambda i: (i, 0)),
        pl.BlockSpec((R, 1), lambda i: (i, 0)),
        pl.BlockSpec((1, D_HID), lambda i: (0, 0)),
        pl.BlockSpec((D_HID, N_CLS), lambda i: (0, 0)),
    ],
    out_specs=pl.BlockSpec((R, D_HID), lambda i: (i, 0)),
    out_shape=jax.ShapeDtypeStruct((NP, D_HID), _f32),
)


def _tc_out_body(parts_ref, deg_i_ref, b2_ref, out_ref):
    ni = _norm(deg_i_ref[...])
    ps = parts_ref[...]
    out_ref[...] = (ps[0, :, :N_CLS] + ps[1, :, :N_CLS]) * ni + b2_ref[...]


_tc_out = pl.pallas_call(
    _tc_out_body,
    grid=(N // R,),
    in_specs=[
        pl.BlockSpec((NC, R, D_HID), lambda i: (0, i, 0)),
        pl.BlockSpec((R, 1), lambda i: (i, 0)),
        pl.BlockSpec((1, N_CLS), lambda i: (0, 0)),
    ],
    out_specs=pl.BlockSpec((R, N_CLS), lambda i: (i, 0)),
    out_shape=jax.ShapeDtypeStruct((N, N_CLS), _f32),
)


def kernel(x, edge_index, W1, b1, W2, b2):
    e128 = edge_index.reshape(2, NR128, 128)
    e64 = edge_index.reshape(2, NR64, 64)
    zeros_n = jnp.zeros((NP,), _f32)
    zeros_nd = jnp.zeros((NP, D_HID), _f32)

    deg = _sc_degrees(e128, e64, zeros_n)
    deg_o = deg[0, :N].reshape(N, 1)
    deg_i = deg[1, :N].reshape(N, 1)

    y1 = _tc_mm(x, W1)
    h1s = _tc_scale(deg_o, y1)
    parts = _mp128(e128, e64, h1s, zeros_nd)
    h2s = _tc_mid(parts, deg_i, deg_o, b1.reshape(1, D_HID), W2)
    parts2 = _mp128(e128, e64, h2s, zeros_nd)
    return _tc_out(parts2, deg_i, b2.reshape(1, N_CLS))
